# baseline (speedup 1.0000x reference)
import jax
import jax.numpy as jnp
from jax.experimental import pallas as pl
from jax.experimental.pallas import tpu as pltpu

LANES = 8192
ROWS = 24
NSUB = 8
NPRI = 2


def _pos_body(xe_ref, ye_ref, pos_ref):
    pos_ref[...] = xe_ref[...][:, None, :] + ye_ref[...][None, :, :]


def _add_body(x_ref, pos_ref, o_ref, ibuf, obuf, isem, osem):
    i = pl.program_id(0)
    nsteps = pl.num_programs(0)
    ph = jax.lax.rem(i, 2)

    def start_ins(step, phase):
        for q in range(NSUB):
            pltpu.make_async_copy(
                x_ref.at[step, q], ibuf.at[phase, q], isem.at[phase]
            ).start(priority=q % NPRI)

    @pl.when(i == 0)
    def _prologue():
        start_ins(0, 0)
        start_ins(1, 1)

    pltpu.make_async_copy(x_ref.at[i], ibuf.at[ph], isem.at[ph]).wait()

    @pl.when(i >= 2)
    def _wait_prev_out():
        pltpu.make_async_copy(
            obuf.at[ph], o_ref.at[i - 2], osem.at[ph]
        ).wait()

    obuf[ph] = ibuf[ph] + pos_ref[...][None]

    for q in range(NSUB):
        pltpu.make_async_copy(
            obuf.at[ph, q], o_ref.at[i, q], osem.at[ph]
        ).start(priority=q % NPRI)

    @pl.when(i + 2 < nsteps)
    def _prefetch():
        start_ins(i + 2, ph)

    @pl.when(i == nsteps - 1)
    def _drain():
        pltpu.make_async_copy(
            obuf.at[0], o_ref.at[nsteps - 2], osem.at[0]
        ).wait()
        pltpu.make_async_copy(
            obuf.at[1], o_ref.at[nsteps - 1], osem.at[1]
        ).wait()


def kernel(x, xemb, yemb):
    B, H, W, D = x.shape

    posemb = pl.pallas_call(
        _pos_body,
        out_shape=jax.ShapeDtypeStruct((H, W, D), x.dtype),
    )(xemb, yemb)

    pos2 = posemb.reshape(ROWS, LANES)
    nsteps = B // NSUB
    x4 = x.reshape(nsteps, NSUB, ROWS, LANES)

    out = pl.pallas_call(
        _add_body,
        grid=(nsteps,),
        in_specs=[
            pl.BlockSpec(memory_space=pltpu.MemorySpace.HBM),
            pl.BlockSpec((ROWS, LANES), lambda i: (0, 0)),
        ],
        out_specs=pl.BlockSpec(memory_space=pltpu.MemorySpace.HBM),
        out_shape=jax.ShapeDtypeStruct((nsteps, NSUB, ROWS, LANES), x.dtype),
        scratch_shapes=[
            pltpu.VMEM((2, NSUB, ROWS, LANES), x.dtype),
            pltpu.VMEM((2, NSUB, ROWS, LANES), x.dtype),
            pltpu.SemaphoreType.DMA((2,)),
            pltpu.SemaphoreType.DMA((2,)),
        ],
        compiler_params=pltpu.CompilerParams(
            dimension_semantics=("arbitrary",),
        ),
    )(x4, pos2)
    return out.reshape(B, H, W, D)


# speedup vs baseline: 1.8765x; 1.8765x over previous
import functools
import jax
import jax.numpy as jnp
from jax import lax
from jax.experimental import pallas as pl
from jax.experimental.pallas import tpu as pltpu
from jax.experimental.pallas import tpu_sc as plsc

L = 16
NC, NS = 2, 16

LANES = 8192
ROWS = 24
BB = 8


def _sc_posemb(xemb, yemb):
    H, D = xemb.shape
    W, _ = yemb.shape
    dv = D // L

    mesh = plsc.VectorSubcoreMesh(
        core_axis_name="c", subcore_axis_name="s", num_cores=NC, num_subcores=NS
    )

    @functools.partial(
        pl.kernel,
        mesh=mesh,
        out_type=jax.ShapeDtypeStruct((H, W, D), jnp.float32),
        scratch_types=[
            pltpu.VMEM((D,), jnp.float32),
            pltpu.VMEM((W, D), jnp.float32),
            pltpu.VMEM((W, D), jnp.float32),
        ],
    )
    def _build(xe_hbm, ye_hbm, pos_hbm, xe_v, ye_v, pos_v):
        wid = lax.axis_index("s") * NC + lax.axis_index("c")
        pltpu.sync_copy(xe_hbm.at[wid], xe_v)
        pltpu.sync_copy(ye_hbm, ye_v)
        xrow = tuple(xe_v[pl.ds(j * L, L)] for j in range(dv))

        @plsc.parallel_loop(0, W, unroll=2, carry=xrow)
        def _w(w, xrow):
            for j in range(dv):
                sl = pl.ds(j * L, L)
                pos_v[w, sl] = ye_v[w, sl] + xrow[j]
            return xrow

        pltpu.sync_copy(pos_v, pos_hbm.at[wid])

    return _build(xemb, yemb)


def _add_body(x_ref, pos_ref, o_ref):
    o_ref[...] = x_ref[...] + pos_ref[...][None]


def kernel(x, xemb, yemb):
    B, H, W, D = x.shape

    posemb = _sc_posemb(xemb, yemb)

    pos2 = posemb.reshape(ROWS, LANES)
    x3 = x.reshape(B, ROWS, LANES)

    out = pl.pallas_call(
        _add_body,
        grid=(B // BB,),
        in_specs=[
            pl.BlockSpec((BB, ROWS, LANES), lambda i: (i, 0, 0)),
            pl.BlockSpec((ROWS, LANES), lambda i: (0, 0)),
        ],
        out_specs=pl.BlockSpec((BB, ROWS, LANES), lambda i: (i, 0, 0)),
        out_shape=jax.ShapeDtypeStruct((B, ROWS, LANES), x.dtype),
    )(x3, pos2)
    return out.reshape(B, H, W, D)


# speedup vs baseline: 1.8929x; 1.0087x over previous
import functools
import jax
import jax.numpy as jnp
from jax import lax
from jax.experimental import pallas as pl
from jax.experimental.pallas import tpu as pltpu
from jax.experimental.pallas import tpu_sc as plsc

L = 16
NC, NS = 2, 16
BB = 8


def _sc_posemb(xemb, ye_flat, W):
    H, D = xemb.shape
    row = ye_flat.shape[0]
    dv = D // L

    mesh = plsc.VectorSubcoreMesh(
        core_axis_name="c", subcore_axis_name="s", num_cores=NC, num_subcores=NS
    )

    @functools.partial(
        pl.kernel,
        mesh=mesh,
        out_type=jax.ShapeDtypeStruct((H, row), jnp.float32),
        scratch_types=[
            pltpu.VMEM((D,), jnp.float32),
            pltpu.VMEM((row,), jnp.float32),
            pltpu.VMEM((row,), jnp.float32),
        ],
    )
    def _build(xe_hbm, ye_hbm, pos_hbm, xe_v, ye_v, pos_v):
        wid = lax.axis_index("s") * NC + lax.axis_index("c")
        pltpu.sync_copy(xe_hbm.at[wid], xe_v)
        pltpu.sync_copy(ye_hbm, ye_v)
        xrow = tuple(xe_v[pl.ds(j * L, L)] for j in range(dv))

        @plsc.parallel_loop(0, W, unroll=2, carry=xrow)
        def _w(w, xrow):
            for j in range(dv):
                sl = pl.ds(w * D + j * L, L)
                pos_v[sl] = ye_v[sl] + xrow[j]
            return xrow

        pltpu.sync_copy(pos_v, pos_hbm.at[wid])

    return _build(xemb, ye_flat)


def _add_body(x_ref, pos_ref, o_ref):
    o_ref[...] = x_ref[...] + pos_ref[...][None]


def kernel(x, xemb, yemb):
    B, H, W, D = x.shape
    row = W * D

    pos2 = _sc_posemb(xemb, yemb.reshape(row), W)
    x3 = x.reshape(B, H, row)

    out = pl.pallas_call(
        _add_body,
        grid=(B // BB,),
        in_specs=[
            pl.BlockSpec((BB, H, row), lambda i: (i, 0, 0)),
            pl.BlockSpec((H, row), lambda i: (0, 0)),
        ],
        out_specs=pl.BlockSpec((BB, H, row), lambda i: (i, 0, 0)),
        out_shape=jax.ShapeDtypeStruct((B, H, row), x.dtype),
    )(x3, pos2)
    return out.reshape(B, H, W, D)


# speedup vs baseline: 1.9086x; 1.0083x over previous
import functools
import jax
import jax.numpy as jnp
from jax import lax
from jax.experimental import pallas as pl
from jax.experimental.pallas import tpu as pltpu
from jax.experimental.pallas import tpu_sc as plsc

L = 16
NC, NS = 2, 16
BB = 16


def _sc_posemb(xemb, ye_flat, W):
    H, D = xemb.shape
    row = ye_flat.shape[0]
    dv = D // L

    mesh = plsc.VectorSubcoreMesh(
        core_axis_name="c", subcore_axis_name="s", num_cores=NC, num_subcores=NS
    )

    @functools.partial(
        pl.kernel,
        mesh=mesh,
        out_type=jax.ShapeDtypeStruct((H, row), jnp.float32),
        scratch_types=[
            pltpu.VMEM((D,), jnp.float32),
            pltpu.VMEM((row,), jnp.float32),
            pltpu.VMEM((row,), jnp.float32),
        ],
    )
    def _build(xe_hbm, ye_hbm, pos_hbm, xe_v, ye_v, pos_v):
        wid = lax.axis_index("s") * NC + lax.axis_index("c")
        pltpu.sync_copy(xe_hbm.at[wid], xe_v)
        pltpu.sync_copy(ye_hbm, ye_v)
        xrow = tuple(xe_v[pl.ds(j * L, L)] for j in range(dv))

        @plsc.parallel_loop(0, W, unroll=2, carry=xrow)
        def _w(w, xrow):
            for j in range(dv):
                sl = pl.ds(w * D + j * L, L)
                pos_v[sl] = ye_v[sl] + xrow[j]
            return xrow

        pltpu.sync_copy(pos_v, pos_hbm.at[wid])

    return _build(xemb, ye_flat)


def _add_body(x_ref, pos_ref, o_ref):
    o_ref[...] = x_ref[...] + pos_ref[...][None]


def kernel(x, xemb, yemb):
    B, H, W, D = x.shape
    row = W * D

    pos2 = _sc_posemb(xemb, yemb.reshape(row), W)
    x3 = x.reshape(B, H, row)

    out = pl.pallas_call(
        _add_body,
        grid=(B // BB,),
        in_specs=[
            pl.BlockSpec((BB, H, row), lambda i: (i, 0, 0)),
            pl.BlockSpec((H, row), lambda i: (0, 0)),
        ],
        out_specs=pl.BlockSpec((BB, H, row), lambda i: (i, 0, 0)),
        out_shape=jax.ShapeDtypeStruct((B, H, row), x.dtype),
    )(x3, pos2)
    return out.reshape(B, H, W, D)
